# Initial kernel scaffold; baseline (speedup 1.0000x reference)
import jax
import jax.numpy as jnp
from jax.experimental import pallas as pl


def kernel(src, ln1_g, ln1_b, ln2_g, ln2_b, Wq, Wk, Wv, Wo, sel_w, keys, values):
    raise NotImplementedError("write your pallas kernel here")


# speedup vs baseline: 1.3957x; 1.3957x over previous
import functools
import math

import jax
import jax.numpy as jnp
from jax.experimental import pallas as pl

D = 1024
H = 16
DH = D // H
NROT = DH // 2
E = 64
F = 128
K = 2
S = 2048
ROPE_BASE = 10000.0
SB = 256
NSB = S // SB


def _ln(x, g, b):
    m = jnp.mean(x, axis=-1, keepdims=True)
    v = jnp.mean((x - m) ** 2, axis=-1, keepdims=True)
    return (x - m) * jax.lax.rsqrt(v + 1e-5) * g + b


def _qkv_body(x_ref, g_ref, b_ref, wq_ref, wk_ref, wv_ref, c_ref, s_ref,
              q_ref, k_ref, v_ref):
    xb = x_ref[...]
    nx = _ln(xb, g_ref[...], b_ref[...])
    q = jnp.dot(nx, wq_ref[...], preferred_element_type=jnp.float32)
    k = jnp.dot(nx, wk_ref[...], preferred_element_type=jnp.float32)
    v = jnp.dot(nx, wv_ref[...], preferred_element_type=jnp.float32)
    cb = c_ref[...]
    sb = s_ref[...]
    cfull = jnp.concatenate([cb] * H, axis=1)
    sfull = jnp.concatenate([sb] * H, axis=1)
    lane = jax.lax.broadcasted_iota(jnp.int32, (SB, D), 1)
    even = (lane % 2) == 0
    qs = jnp.where(even, jnp.roll(q, -1, axis=1), jnp.roll(q, 1, axis=1))
    ks = jnp.where(even, jnp.roll(k, -1, axis=1), jnp.roll(k, 1, axis=1))
    q_ref[...] = q * cfull + qs * sfull
    k_ref[...] = k * cfull + ks * sfull
    v_ref[...] = v


def _attn_body(q_ref, k_ref, v_ref, o_ref):
    qb = q_ref[...]
    kb = k_ref[...]
    vb = v_ref[...]
    outs = []
    for h in range(H):
        qh = qb[:, h * DH:(h + 1) * DH]
        kh = kb[:, h * DH:(h + 1) * DH]
        vh = vb[:, h * DH:(h + 1) * DH]
        s = jax.lax.dot_general(qh, kh, (((1,), (1,)), ((), ())),
                                preferred_element_type=jnp.float32)
        s = s * (1.0 / math.sqrt(DH))
        m = jnp.max(s, axis=1, keepdims=True)
        p = jnp.exp(s - m)
        denom = jnp.sum(p, axis=1, keepdims=True)
        o = jnp.dot(p, vh, preferred_element_type=jnp.float32)
        outs.append(o / denom)
    o_ref[...] = jnp.concatenate(outs, axis=1)


def _proj_router_body(o_ref, src_ref, wo_ref, g2_ref, b2_ref, selw_ref,
                      x_ref, x2_ref, gate_ref):
    xb = src_ref[...] + jnp.dot(o_ref[...], wo_ref[...],
                                preferred_element_type=jnp.float32)
    x_ref[...] = xb
    nx = _ln(xb, g2_ref[...], b2_ref[...])
    x2_ref[...] = nx
    logits = jnp.dot(nx, selw_ref[...], preferred_element_type=jnp.float32)
    sel = jax.nn.sigmoid(logits)
    iota = jax.lax.broadcasted_iota(jnp.int32, (SB, E), 1)
    m1 = jnp.max(sel, axis=1, keepdims=True)
    i1 = jnp.min(jnp.where(sel == m1, iota, E), axis=1, keepdims=True)
    masked = jnp.where(iota == i1, -jnp.inf, sel)
    m2 = jnp.max(masked, axis=1, keepdims=True)
    i2 = jnp.min(jnp.where(masked == m2, iota, E), axis=1, keepdims=True)
    gate = jnp.where(iota == i1, m1, 0.0) + jnp.where(iota == i2, m2, 0.0)
    gate_ref[...] = gate


def _moe_body(x_ref, x2_ref, gate_ref, keys_ref, values_ref, out_ref):
    e = pl.program_id(0)

    @pl.when(e == 0)
    def _():
        out_ref[...] = x_ref[...]

    onehot = (jax.lax.broadcasted_iota(jnp.int32, (E, 1), 0) == e
              ).astype(jnp.float32)
    g = jnp.dot(gate_ref[...], onehot,
                preferred_element_type=jnp.float32)
    h = jnp.dot(x2_ref[...], keys_ref[0], preferred_element_type=jnp.float32)
    h = jnp.maximum(h, 0.0) * g
    out_ref[...] += jnp.dot(h, values_ref[0],
                            preferred_element_type=jnp.float32)


def _rope_tables():
    pos = jnp.arange(S, dtype=jnp.float32)
    half = NROT // 2
    inv_freq = ROPE_BASE ** (-jnp.arange(half, dtype=jnp.float32) / half)
    ang = pos[:, None] * inv_freq[None, :]
    cos = jnp.repeat(jnp.cos(ang), 2, axis=1)
    sin = jnp.repeat(jnp.sin(ang), 2, axis=1)
    sign = jnp.where(jnp.arange(NROT) % 2 == 0, -1.0, 1.0)
    c = jnp.concatenate([cos, jnp.ones((S, DH - NROT))], axis=1)
    s = jnp.concatenate([sin * sign, jnp.zeros((S, DH - NROT))], axis=1)
    return c.astype(jnp.float32), s.astype(jnp.float32)


@jax.jit
def kernel(src, ln1_g, ln1_b, ln2_g, ln2_b, Wq, Wk, Wv, Wo, sel_w, keys,
           values):
    x0 = src.reshape(S, D)
    ctab, stab = _rope_tables()
    g1 = ln1_g.reshape(1, D)
    b1 = ln1_b.reshape(1, D)
    g2 = ln2_g.reshape(1, D)
    b2 = ln2_b.reshape(1, D)

    full = pl.BlockSpec((D, D), lambda i: (0, 0))
    row = pl.BlockSpec((1, D), lambda i: (0, 0))
    sblk = pl.BlockSpec((SB, D), lambda i: (i, 0))
    rblk = pl.BlockSpec((SB, DH), lambda i: (i, 0))

    q, k, v = pl.pallas_call(
        _qkv_body,
        grid=(NSB,),
        in_specs=[sblk, row, row, full, full, full, rblk, rblk],
        out_specs=[sblk, sblk, sblk],
        out_shape=[jax.ShapeDtypeStruct((S, D), jnp.float32)] * 3,
    )(x0, g1, b1, Wq, Wk, Wv, ctab, stab)

    o = pl.pallas_call(
        _attn_body,
        grid=(NSB,),
        in_specs=[
            sblk,
            pl.BlockSpec((S, D), lambda i: (0, 0)),
            pl.BlockSpec((S, D), lambda i: (0, 0)),
        ],
        out_specs=sblk,
        out_shape=jax.ShapeDtypeStruct((S, D), jnp.float32),
    )(q, k, v)

    x, x2, gate = pl.pallas_call(
        _proj_router_body,
        grid=(NSB,),
        in_specs=[sblk, sblk, full, row, row,
                  pl.BlockSpec((D, E), lambda i: (0, 0))],
        out_specs=[sblk, sblk, pl.BlockSpec((SB, E), lambda i: (i, 0))],
        out_shape=[
            jax.ShapeDtypeStruct((S, D), jnp.float32),
            jax.ShapeDtypeStruct((S, D), jnp.float32),
            jax.ShapeDtypeStruct((S, E), jnp.float32),
        ],
    )(o, x0, Wo, g2, b2, sel_w)

    out = pl.pallas_call(
        _moe_body,
        grid=(E,),
        in_specs=[
            pl.BlockSpec((S, D), lambda e: (0, 0)),
            pl.BlockSpec((S, D), lambda e: (0, 0)),
            pl.BlockSpec((S, E), lambda e: (0, 0)),
            pl.BlockSpec((1, D, F), lambda e: (e, 0, 0)),
            pl.BlockSpec((1, F, D), lambda e: (e, 0, 0)),
        ],
        out_specs=pl.BlockSpec((S, D), lambda e: (0, 0)),
        out_shape=jax.ShapeDtypeStruct((S, D), jnp.float32),
    )(x, x2, gate, keys, values)

    return out.reshape(1, S, D)
